# Initial kernel scaffold; baseline (speedup 1.0000x reference)
import jax
import jax.numpy as jnp
from jax.experimental import pallas as pl


def kernel(patient, treatment, confounders, corpus_embeddings, Wpe, bpe, W1, b1, g1, be1, W2, b2, g2, be2, W3, b3, Wo1, bo1, Wo2, bo2, Wt1, bt1, Wt2, bt2, Wg1, bg1, Wg2, bg2):
    raise NotImplementedError("write your pallas kernel here")


# speedup vs baseline: 1.3609x; 1.3609x over previous
import jax
import jax.numpy as jnp
from jax import lax
from jax.experimental import pallas as pl
from jax.experimental.pallas import tpu as pltpu
from jax.experimental.pallas import tpu_sc as plsc

_B = 128
_NDOCS = 1000000
_EMB = 64
_K = 8
_BLK = 8192
_NBLK = (_NDOCS + _BLK - 1) // _BLK

_NEG_INF = float("-inf")
_IMAX = 2147483647

_SC_NC = 2
_SC_NS = 16
_SC_NW = _SC_NC * _SC_NS


def _dotT(a, w, precision=None):
    return lax.dot_general(a, w, (((1,), (1,)), ((), ())),
                           preferred_element_type=jnp.float32,
                           precision=precision)


def _topk_body(patient_ref, wpe_ref, bpe_ref, corpus_ref,
               scores_out, idx_out, pemb_s, runs_s, runi_s):
    i = pl.program_id(0)

    @pl.when(i == 0)
    def _init():
        pe = _dotT(patient_ref[...], wpe_ref[...]) + bpe_ref[...]
        n = jnp.sqrt(jnp.sum(pe * pe, axis=1, keepdims=True))
        pemb_s[...] = pe / jnp.maximum(n, 1e-12)
        runs_s[...] = jnp.full((_B, _K), _NEG_INF, jnp.float32)
        runi_s[...] = jnp.zeros((_B, _K), jnp.int32)

    c = corpus_ref[...]
    n = jnp.sqrt(jnp.sum(c * c, axis=1, keepdims=True))
    cn = c / jnp.maximum(n, 1e-12)
    sims = _dotT(pemb_s[...], cn)
    gidx = lax.broadcasted_iota(jnp.int32, (_B, _BLK), 1) + i * _BLK
    sims = jnp.where(gidx < _NDOCS, sims, _NEG_INF)

    cand = jnp.concatenate([sims, runs_s[...]], axis=1)
    cidx = jnp.concatenate([gidx, runi_s[...]], axis=1)

    ms, sels = [], []
    for _ in range(_K):
        m = jnp.max(cand, axis=1, keepdims=True)
        eq = cand == m
        sel = jnp.min(jnp.where(eq, cidx, _IMAX), axis=1, keepdims=True)
        ms.append(m)
        sels.append(sel)
        cand = jnp.where(eq & (cidx == sel), _NEG_INF, cand)
    runs_s[...] = jnp.concatenate(ms, axis=1)
    runi_s[...] = jnp.concatenate(sels, axis=1)

    @pl.when(i == _NBLK - 1)
    def _fin():
        scores_out[...] = runs_s[...]
        idx_out[...] = runi_s[...]


def _topk_call(patient, Wpe, bpe2d, corpus):
    return pl.pallas_call(
        _topk_body,
        grid=(_NBLK,),
        in_specs=[
            pl.BlockSpec((_B, patient.shape[1]), lambda i: (0, 0)),
            pl.BlockSpec(Wpe.shape, lambda i: (0, 0)),
            pl.BlockSpec(bpe2d.shape, lambda i: (0, 0)),
            pl.BlockSpec((_BLK, _EMB), lambda i: (i, 0)),
        ],
        out_specs=[
            pl.BlockSpec((_B, _K), lambda i: (0, 0)),
            pl.BlockSpec((_B, _K), lambda i: (0, 0)),
        ],
        out_shape=[
            jax.ShapeDtypeStruct((_B, _K), jnp.float32),
            jax.ShapeDtypeStruct((_B, _K), jnp.int32),
        ],
        scratch_shapes=[
            pltpu.VMEM((_B, _EMB), jnp.float32),
            pltpu.VMEM((_B, _K), jnp.float32),
            pltpu.VMEM((_B, _K), jnp.int32),
        ],
        compiler_params=pltpu.CompilerParams(
            dimension_semantics=("arbitrary",)),
    )(patient, Wpe, bpe2d, corpus)


def _gather_body(table_hbm, idx_hbm, out_hbm, idx_v, rows_v, sem):
    wid = lax.axis_index("s") * _SC_NC + lax.axis_index("c")
    bpw = (_B * _K) // _SC_NW
    base = wid * bpw
    pltpu.sync_copy(idx_hbm.at[pl.ds(base, bpw)], idx_v)
    pltpu.async_copy(table_hbm.at[idx_v], rows_v, sem).wait()
    pltpu.sync_copy(rows_v, out_hbm.at[pl.ds(base, bpw)])


def _gather_call(corpus_pairs, idx2_flat):
    bpw = (_B * _K) // _SC_NW
    mesh = plsc.VectorSubcoreMesh(core_axis_name="c", subcore_axis_name="s")
    k = pl.kernel(
        _gather_body,
        mesh=mesh,
        out_type=jax.ShapeDtypeStruct((_B * _K, 2 * _EMB), jnp.float32),
        scratch_types=[
            pltpu.VMEM((bpw,), jnp.int32),
            pltpu.VMEM((bpw, 2 * _EMB), jnp.float32),
            pltpu.SemaphoreType.DMA,
        ],
    )
    return k(corpus_pairs, idx2_flat)


def _mlp_body(conf_ref, pairs_ref, par_ref, treat_ref,
              w1_ref, b1_ref, g1_ref, be1_ref,
              w2_ref, b2_ref, g2_ref, be2_ref,
              w3_ref, b3_ref,
              wo1_ref, bo1_ref, wo2_ref, bo2_ref,
              wt1_ref, bt1_ref, wt2_ref, bt2_ref,
              wg1_ref, bg1_ref, wg2_ref, bg2_ref,
              fact_out, prop_out, targ_out, cf_out):
    def ln(x, g, b):
        m = jnp.mean(x, axis=1, keepdims=True)
        v = jnp.mean((x - m) ** 2, axis=1, keepdims=True)
        return (x - m) / jnp.sqrt(v + 1e-5) * g + b

    pairs = pairs_ref[...]
    par = par_ref[...]
    parts = []
    for k in range(_K):
        seg = pairs[:, k * 2 * _EMB:(k + 1) * 2 * _EMB]
        m = par[:, k:k + 1] == 0
        parts.append(jnp.where(m, seg[:, :_EMB], seg[:, _EMB:]))

    x = jnp.concatenate([conf_ref[...]] + parts, axis=1)
    h = ln(jax.nn.relu(_dotT(x, w1_ref[...]) + b1_ref[...]),
           g1_ref[...], be1_ref[...])
    h = ln(jax.nn.relu(_dotT(h, w2_ref[...]) + b2_ref[...]),
           g2_ref[...], be2_ref[...])
    shared = _dotT(h, w3_ref[...]) + b3_ref[...]

    t1 = jax.nn.relu(_dotT(shared, wt1_ref[...]) + bt1_ref[...])
    logits = _dotT(t1, wt2_ref[...]) + bt2_ref[...]
    lm = jnp.max(logits, axis=1, keepdims=True)
    e = jnp.exp(logits - lm)
    prop_out[...] = e / jnp.sum(e, axis=1, keepdims=True)

    def out_head(tvec, w_a, b_a, w_b_row, b_b_scalar):
        ci = jnp.concatenate([shared, tvec], axis=1)
        o1 = jax.nn.relu(_dotT(ci, w_a) + b_a)
        return jnp.sum(o1 * w_b_row, axis=1, keepdims=True) + b_b_scalar

    bo2_s = bo2_ref[0]
    bg2_s = bg2_ref[0]
    fact_out[...] = out_head(treat_ref[...], wo1_ref[...], bo1_ref[...],
                             wo2_ref[...], bo2_s)
    targ_out[...] = out_head(treat_ref[...], wg1_ref[...], bg1_ref[...],
                             wg2_ref[...], bg2_s)
    ones = jnp.ones((_B, 1), jnp.float32)
    zeros = jnp.zeros((_B, 1), jnp.float32)
    cf0 = out_head(jnp.concatenate([ones, zeros], axis=1),
                   wo1_ref[...], bo1_ref[...], wo2_ref[...], bo2_s)
    cf1 = out_head(jnp.concatenate([zeros, ones], axis=1),
                   wo1_ref[...], bo1_ref[...], wo2_ref[...], bo2_s)
    cf_out[...] = jnp.concatenate([cf0, cf1], axis=1)


def _mlp_call(conf, pairs, par, treat, weights):
    args = (conf, pairs, par, treat) + tuple(weights)
    specs = []
    for a in args:
        if a.ndim == 1:
            specs.append(pl.BlockSpec(memory_space=pltpu.SMEM))
        else:
            specs.append(pl.BlockSpec(memory_space=pltpu.VMEM))
    return pl.pallas_call(
        _mlp_body,
        in_specs=specs,
        out_shape=[
            jax.ShapeDtypeStruct((_B, 1), jnp.float32),
            jax.ShapeDtypeStruct((_B, 2), jnp.float32),
            jax.ShapeDtypeStruct((_B, 1), jnp.float32),
            jax.ShapeDtypeStruct((_B, 2), jnp.float32),
        ],
    )(conf, pairs, par, treat, *weights)


def kernel(patient, treatment, confounders, corpus_embeddings, Wpe, bpe,
           W1, b1, g1, be1, W2, b2, g2, be2, W3, b3,
           Wo1, bo1, Wo2, bo2, Wt1, bt1, Wt2, bt2, Wg1, bg1, Wg2, bg2):
    row = lambda v: v.reshape(1, -1)
    scores, idx = _topk_call(patient, Wpe, row(bpe), corpus_embeddings)
    idx_flat = idx.reshape(_B * _K)
    corpus_pairs = corpus_embeddings.reshape(_NDOCS // 2, 2 * _EMB)
    gathered = _gather_call(corpus_pairs, idx_flat // 2)
    pairs = gathered.reshape(_B, _K * 2 * _EMB)
    par = idx % 2
    weights = (W1, row(b1), row(g1), row(be1),
               W2, row(b2), row(g2), row(be2),
               W3, row(b3),
               Wo1, row(bo1), Wo2, bo2,
               Wt1, row(bt1), Wt2, row(bt2),
               Wg1, row(bg1), Wg2, bg2)
    factual, propensity, targeted, cf = _mlp_call(
        confounders, pairs, par, treatment, weights)
    counterfactuals = cf.reshape(_B, 2, 1)
    return (factual, propensity, targeted, counterfactuals, scores, idx)


# speedup vs baseline: 2.0716x; 1.5222x over previous
import jax
import jax.numpy as jnp
from jax import lax
from jax.experimental import pallas as pl
from jax.experimental.pallas import tpu as pltpu
from jax.experimental.pallas import tpu_sc as plsc

_B = 128
_NDOCS = 1000000
_EMB = 64
_K = 8
_BLK = 8192
_NBLK = (_NDOCS + _BLK - 1) // _BLK

_NEG_INF = float("-inf")
_IMAX = 2147483647

_SC_NC = 2
_SC_NS = 16
_SC_NW = _SC_NC * _SC_NS


def _dotT(a, w, precision=None):
    return lax.dot_general(a, w, (((1,), (1,)), ((), ())),
                           preferred_element_type=jnp.float32,
                           precision=precision)


_CH = 128
_NCH = _BLK // _CH


def _extract8(cand, candidx, axis):
    ms, sels = [], []
    for _ in range(_K):
        m = jnp.max(cand, axis=axis, keepdims=True)
        eq = cand == m
        sel = jnp.min(jnp.where(eq, candidx, _IMAX), axis=axis, keepdims=True)
        ms.append(m)
        sels.append(sel)
        cand = jnp.where(eq & (candidx == sel), _NEG_INF, cand)
    return jnp.concatenate(ms, axis=axis), jnp.concatenate(sels, axis=axis)


def _topk_body(patient_ref, wpe_ref, bpe_ref, corpus_ref,
               scores_out, idx_out, pemb_s, runs_s, runi_s):
    i = pl.program_id(0)

    @pl.when(i == 0)
    def _init():
        pe = _dotT(patient_ref[...], wpe_ref[...]) + bpe_ref[...]
        n = jnp.sqrt(jnp.sum(pe * pe, axis=1, keepdims=True))
        pemb_s[...] = pe / jnp.maximum(n, 1e-12)
        runs_s[...] = jnp.full((_K, _B), _NEG_INF, jnp.float32)
        runi_s[...] = jnp.zeros((_K, _B), jnp.int32)

    c = corpus_ref[...]
    n = jnp.sqrt(jnp.sum(c * c, axis=1, keepdims=True))
    cn = c / jnp.maximum(n, 1e-12)
    sims = _dotT(cn, pemb_s[...])
    dmask = (i * _BLK + lax.broadcasted_iota(jnp.int32, (_BLK, _B), 0)) < _NDOCS
    sims = jnp.where(dmask, sims, _NEG_INF)

    old_s = runs_s[...]
    old_i = runi_s[...]

    x3 = sims.reshape(_NCH, _CH, _B)
    cmax = jnp.max(x3, axis=1)
    eqc = x3 == cmax[:, None, :]
    off3 = lax.broadcasted_iota(jnp.int32, (_NCH, _CH, _B), 1)
    coff = jnp.min(jnp.where(eqc, off3, _CH), axis=1)
    cidx = i * _BLK + lax.broadcasted_iota(jnp.int32, (_NCH, _B), 0) * _CH + coff

    new_s, new_i = _extract8(jnp.concatenate([cmax, old_s], axis=0),
                             jnp.concatenate([cidx, old_i], axis=0), 0)

    t8 = new_s[_K - 1:_K, :]
    cnt = jnp.sum((sims >= t8).astype(jnp.int32), axis=0, keepdims=True)
    sb = jnp.sum((new_i >= i * _BLK).astype(jnp.int32), axis=0, keepdims=True)
    ok = jnp.all(cnt <= sb)

    runs_s[...] = new_s
    runi_s[...] = new_i

    @pl.when(jnp.logical_not(ok))
    def _fallback():
        gidx = i * _BLK + lax.broadcasted_iota(jnp.int32, (_BLK, _B), 0)
        fs, fi = _extract8(jnp.concatenate([sims, old_s], axis=0),
                           jnp.concatenate([gidx, old_i], axis=0), 0)
        runs_s[...] = fs
        runi_s[...] = fi

    @pl.when(i == _NBLK - 1)
    def _fin():
        scores_out[...] = runs_s[...]
        idx_out[...] = runi_s[...]


def _topk_call(patient, Wpe, bpe2d, corpus):
    return pl.pallas_call(
        _topk_body,
        grid=(_NBLK,),
        in_specs=[
            pl.BlockSpec((_B, patient.shape[1]), lambda i: (0, 0)),
            pl.BlockSpec(Wpe.shape, lambda i: (0, 0)),
            pl.BlockSpec(bpe2d.shape, lambda i: (0, 0)),
            pl.BlockSpec((_BLK, _EMB), lambda i: (i, 0)),
        ],
        out_specs=[
            pl.BlockSpec((_K, _B), lambda i: (0, 0)),
            pl.BlockSpec((_K, _B), lambda i: (0, 0)),
        ],
        out_shape=[
            jax.ShapeDtypeStruct((_K, _B), jnp.float32),
            jax.ShapeDtypeStruct((_K, _B), jnp.int32),
        ],
        scratch_shapes=[
            pltpu.VMEM((_B, _EMB), jnp.float32),
            pltpu.VMEM((_K, _B), jnp.float32),
            pltpu.VMEM((_K, _B), jnp.int32),
        ],
        compiler_params=pltpu.CompilerParams(
            dimension_semantics=("arbitrary",)),
    )(patient, Wpe, bpe2d, corpus)


def _gather_body(table_hbm, idx_hbm, out_hbm, idx_v, rows_v, sem):
    wid = lax.axis_index("s") * _SC_NC + lax.axis_index("c")
    bpw = (_B * _K) // _SC_NW
    base = wid * bpw
    pltpu.sync_copy(idx_hbm.at[pl.ds(base, bpw)], idx_v)
    pltpu.async_copy(table_hbm.at[idx_v], rows_v, sem).wait()
    pltpu.sync_copy(rows_v, out_hbm.at[pl.ds(base, bpw)])


def _gather_call(corpus_pairs, idx2_flat):
    bpw = (_B * _K) // _SC_NW
    mesh = plsc.VectorSubcoreMesh(core_axis_name="c", subcore_axis_name="s")
    k = pl.kernel(
        _gather_body,
        mesh=mesh,
        out_type=jax.ShapeDtypeStruct((_B * _K, 2 * _EMB), jnp.float32),
        scratch_types=[
            pltpu.VMEM((bpw,), jnp.int32),
            pltpu.VMEM((bpw, 2 * _EMB), jnp.float32),
            pltpu.SemaphoreType.DMA,
        ],
    )
    return k(corpus_pairs, idx2_flat)


def _mlp_body(conf_ref, pairs_ref, par_ref, treat_ref,
              w1_ref, b1_ref, g1_ref, be1_ref,
              w2_ref, b2_ref, g2_ref, be2_ref,
              w3_ref, b3_ref,
              wo1_ref, bo1_ref, wo2_ref, bo2_ref,
              wt1_ref, bt1_ref, wt2_ref, bt2_ref,
              wg1_ref, bg1_ref, wg2_ref, bg2_ref,
              fact_out, prop_out, targ_out, cf_out):
    def ln(x, g, b):
        m = jnp.mean(x, axis=1, keepdims=True)
        v = jnp.mean((x - m) ** 2, axis=1, keepdims=True)
        return (x - m) / jnp.sqrt(v + 1e-5) * g + b

    pairs = pairs_ref[...]
    par = par_ref[...]
    parts = []
    for k in range(_K):
        seg = pairs[:, k * 2 * _EMB:(k + 1) * 2 * _EMB]
        m = par[:, k:k + 1] == 0
        parts.append(jnp.where(m, seg[:, :_EMB], seg[:, _EMB:]))

    x = jnp.concatenate([conf_ref[...]] + parts, axis=1)
    h = ln(jax.nn.relu(_dotT(x, w1_ref[...]) + b1_ref[...]),
           g1_ref[...], be1_ref[...])
    h = ln(jax.nn.relu(_dotT(h, w2_ref[...]) + b2_ref[...]),
           g2_ref[...], be2_ref[...])
    shared = _dotT(h, w3_ref[...]) + b3_ref[...]

    t1 = jax.nn.relu(_dotT(shared, wt1_ref[...]) + bt1_ref[...])
    logits = _dotT(t1, wt2_ref[...]) + bt2_ref[...]
    lm = jnp.max(logits, axis=1, keepdims=True)
    e = jnp.exp(logits - lm)
    prop_out[...] = e / jnp.sum(e, axis=1, keepdims=True)

    def out_head(tvec, w_a, b_a, w_b_row, b_b_scalar):
        ci = jnp.concatenate([shared, tvec], axis=1)
        o1 = jax.nn.relu(_dotT(ci, w_a) + b_a)
        return jnp.sum(o1 * w_b_row, axis=1, keepdims=True) + b_b_scalar

    bo2_s = bo2_ref[0]
    bg2_s = bg2_ref[0]
    fact_out[...] = out_head(treat_ref[...], wo1_ref[...], bo1_ref[...],
                             wo2_ref[...], bo2_s)
    targ_out[...] = out_head(treat_ref[...], wg1_ref[...], bg1_ref[...],
                             wg2_ref[...], bg2_s)
    ones = jnp.ones((_B, 1), jnp.float32)
    zeros = jnp.zeros((_B, 1), jnp.float32)
    cf0 = out_head(jnp.concatenate([ones, zeros], axis=1),
                   wo1_ref[...], bo1_ref[...], wo2_ref[...], bo2_s)
    cf1 = out_head(jnp.concatenate([zeros, ones], axis=1),
                   wo1_ref[...], bo1_ref[...], wo2_ref[...], bo2_s)
    cf_out[...] = jnp.concatenate([cf0, cf1], axis=1)


def _mlp_call(conf, pairs, par, treat, weights):
    args = (conf, pairs, par, treat) + tuple(weights)
    specs = []
    for a in args:
        if a.ndim == 1:
            specs.append(pl.BlockSpec(memory_space=pltpu.SMEM))
        else:
            specs.append(pl.BlockSpec(memory_space=pltpu.VMEM))
    return pl.pallas_call(
        _mlp_body,
        in_specs=specs,
        out_shape=[
            jax.ShapeDtypeStruct((_B, 1), jnp.float32),
            jax.ShapeDtypeStruct((_B, 2), jnp.float32),
            jax.ShapeDtypeStruct((_B, 1), jnp.float32),
            jax.ShapeDtypeStruct((_B, 2), jnp.float32),
        ],
    )(conf, pairs, par, treat, *weights)


def kernel(patient, treatment, confounders, corpus_embeddings, Wpe, bpe,
           W1, b1, g1, be1, W2, b2, g2, be2, W3, b3,
           Wo1, bo1, Wo2, bo2, Wt1, bt1, Wt2, bt2, Wg1, bg1, Wg2, bg2):
    row = lambda v: v.reshape(1, -1)
    scores_t, idx_t = _topk_call(patient, Wpe, row(bpe), corpus_embeddings)
    scores, idx = scores_t.T, idx_t.T
    idx_flat = idx.reshape(_B * _K)
    corpus_pairs = corpus_embeddings.reshape(_NDOCS // 2, 2 * _EMB)
    gathered = _gather_call(corpus_pairs, idx_flat // 2)
    pairs = gathered.reshape(_B, _K * 2 * _EMB)
    par = idx % 2
    weights = (W1, row(b1), row(g1), row(be1),
               W2, row(b2), row(g2), row(be2),
               W3, row(b3),
               Wo1, row(bo1), Wo2, bo2,
               Wt1, row(bt1), Wt2, row(bt2),
               Wg1, row(bg1), Wg2, bg2)
    factual, propensity, targeted, cf = _mlp_call(
        confounders, pairs, par, treatment, weights)
    counterfactuals = cf.reshape(_B, 2, 1)
    return (factual, propensity, targeted, counterfactuals, scores, idx)


# speedup vs baseline: 2.1276x; 1.0271x over previous
import jax
import jax.numpy as jnp
from jax import lax
from jax.experimental import pallas as pl
from jax.experimental.pallas import tpu as pltpu
from jax.experimental.pallas import tpu_sc as plsc

_B = 128
_NDOCS = 1000000
_EMB = 64
_K = 8
_BLK = 10000
_NBLK = _NDOCS // _BLK

_NEG_INF = float("-inf")
_IMAX = 2147483647

_SC_NC = 2
_SC_NS = 16
_SC_NW = _SC_NC * _SC_NS


def _dotT(a, w, precision=None):
    return lax.dot_general(a, w, (((1,), (1,)), ((), ())),
                           preferred_element_type=jnp.float32,
                           precision=precision)


_CH = 200
_NCH = _BLK // _CH


def _extract8(cand, candidx, axis):
    ms, sels = [], []
    for _ in range(_K):
        m = jnp.max(cand, axis=axis, keepdims=True)
        eq = cand == m
        sel = jnp.min(jnp.where(eq, candidx, _IMAX), axis=axis, keepdims=True)
        ms.append(m)
        sels.append(sel)
        cand = jnp.where(eq & (candidx == sel), _NEG_INF, cand)
    return jnp.concatenate(ms, axis=axis), jnp.concatenate(sels, axis=axis)


def _topk_body(patient_ref, wpe_ref, bpe_ref, corpus_ref,
               scores_out, idx_out, pemb_s, runs_s, runi_s):
    i = pl.program_id(0)

    @pl.when(i == 0)
    def _init():
        pe = _dotT(patient_ref[...], wpe_ref[...]) + bpe_ref[...]
        n = jnp.sqrt(jnp.sum(pe * pe, axis=1, keepdims=True))
        pemb_s[...] = pe / jnp.maximum(n, 1e-12)
        runs_s[...] = jnp.full((_K, _B), _NEG_INF, jnp.float32)
        runi_s[...] = jnp.zeros((_K, _B), jnp.int32)

    c = corpus_ref[...]
    n = jnp.sqrt(jnp.sum(c * c, axis=1, keepdims=True))
    cn = c / jnp.maximum(n, 1e-12)
    sims = _dotT(cn, pemb_s[...])

    old_s = runs_s[...]
    old_i = runi_s[...]

    x3 = sims.reshape(_NCH, _CH, _B)
    cmax = jnp.max(x3, axis=1)
    eqc = x3 == cmax[:, None, :]
    off3 = lax.broadcasted_iota(jnp.int32, (_NCH, _CH, _B), 1)
    coff = jnp.min(jnp.where(eqc, off3, _CH), axis=1)
    cidx = i * _BLK + lax.broadcasted_iota(jnp.int32, (_NCH, _B), 0) * _CH + coff

    new_s, new_i = _extract8(jnp.concatenate([cmax, old_s], axis=0),
                             jnp.concatenate([cidx, old_i], axis=0), 0)

    t8 = new_s[_K - 1:_K, :]
    cnt = jnp.sum((sims >= t8).astype(jnp.int32), axis=0, keepdims=True)
    sb = jnp.sum((new_i >= i * _BLK).astype(jnp.int32), axis=0, keepdims=True)
    ok = jnp.all(cnt <= sb)

    runs_s[...] = new_s
    runi_s[...] = new_i

    @pl.when(jnp.logical_not(ok))
    def _fallback():
        gidx = i * _BLK + lax.broadcasted_iota(jnp.int32, (_BLK, _B), 0)
        fs, fi = _extract8(jnp.concatenate([sims, old_s], axis=0),
                           jnp.concatenate([gidx, old_i], axis=0), 0)
        runs_s[...] = fs
        runi_s[...] = fi

    @pl.when(i == _NBLK - 1)
    def _fin():
        scores_out[...] = runs_s[...]
        idx_out[...] = runi_s[...]


def _topk_call(patient, Wpe, bpe2d, corpus):
    return pl.pallas_call(
        _topk_body,
        grid=(_NBLK,),
        in_specs=[
            pl.BlockSpec((_B, patient.shape[1]), lambda i: (0, 0)),
            pl.BlockSpec(Wpe.shape, lambda i: (0, 0)),
            pl.BlockSpec(bpe2d.shape, lambda i: (0, 0)),
            pl.BlockSpec((_BLK, _EMB), lambda i: (i, 0)),
        ],
        out_specs=[
            pl.BlockSpec((_K, _B), lambda i: (0, 0)),
            pl.BlockSpec((_K, _B), lambda i: (0, 0)),
        ],
        out_shape=[
            jax.ShapeDtypeStruct((_K, _B), jnp.float32),
            jax.ShapeDtypeStruct((_K, _B), jnp.int32),
        ],
        scratch_shapes=[
            pltpu.VMEM((_B, _EMB), jnp.float32),
            pltpu.VMEM((_K, _B), jnp.float32),
            pltpu.VMEM((_K, _B), jnp.int32),
        ],
        compiler_params=pltpu.CompilerParams(
            dimension_semantics=("arbitrary",)),
    )(patient, Wpe, bpe2d, corpus)


def _gather_body(table_hbm, idx_hbm, out_hbm, idx_v, rows_v, sem):
    wid = lax.axis_index("s") * _SC_NC + lax.axis_index("c")
    bpw = (_B * _K) // _SC_NW
    base = wid * bpw
    pltpu.sync_copy(idx_hbm.at[pl.ds(base, bpw)], idx_v)
    pltpu.async_copy(table_hbm.at[idx_v], rows_v, sem).wait()
    pltpu.sync_copy(rows_v, out_hbm.at[pl.ds(base, bpw)])


def _gather_call(corpus_pairs, idx2_flat):
    bpw = (_B * _K) // _SC_NW
    mesh = plsc.VectorSubcoreMesh(core_axis_name="c", subcore_axis_name="s")
    k = pl.kernel(
        _gather_body,
        mesh=mesh,
        out_type=jax.ShapeDtypeStruct((_B * _K, 2 * _EMB), jnp.float32),
        scratch_types=[
            pltpu.VMEM((bpw,), jnp.int32),
            pltpu.VMEM((bpw, 2 * _EMB), jnp.float32),
            pltpu.SemaphoreType.DMA,
        ],
    )
    return k(corpus_pairs, idx2_flat)


def _mlp_body(conf_ref, pairs_ref, par_ref, treat_ref,
              w1_ref, b1_ref, g1_ref, be1_ref,
              w2_ref, b2_ref, g2_ref, be2_ref,
              w3_ref, b3_ref,
              wo1_ref, bo1_ref, wo2_ref, bo2_ref,
              wt1_ref, bt1_ref, wt2_ref, bt2_ref,
              wg1_ref, bg1_ref, wg2_ref, bg2_ref,
              fact_out, prop_out, targ_out, cf_out):
    def ln(x, g, b):
        m = jnp.mean(x, axis=1, keepdims=True)
        v = jnp.mean((x - m) ** 2, axis=1, keepdims=True)
        return (x - m) / jnp.sqrt(v + 1e-5) * g + b

    pairs = pairs_ref[...]
    par = par_ref[...]
    parts = []
    for k in range(_K):
        seg = pairs[:, k * 2 * _EMB:(k + 1) * 2 * _EMB]
        m = par[:, k:k + 1] == 0
        parts.append(jnp.where(m, seg[:, :_EMB], seg[:, _EMB:]))

    x = jnp.concatenate([conf_ref[...]] + parts, axis=1)
    h = ln(jax.nn.relu(_dotT(x, w1_ref[...]) + b1_ref[...]),
           g1_ref[...], be1_ref[...])
    h = ln(jax.nn.relu(_dotT(h, w2_ref[...]) + b2_ref[...]),
           g2_ref[...], be2_ref[...])
    shared = _dotT(h, w3_ref[...]) + b3_ref[...]

    t1 = jax.nn.relu(_dotT(shared, wt1_ref[...]) + bt1_ref[...])
    logits = _dotT(t1, wt2_ref[...]) + bt2_ref[...]
    lm = jnp.max(logits, axis=1, keepdims=True)
    e = jnp.exp(logits - lm)
    prop_out[...] = e / jnp.sum(e, axis=1, keepdims=True)

    def out_head(tvec, w_a, b_a, w_b_row, b_b_scalar):
        ci = jnp.concatenate([shared, tvec], axis=1)
        o1 = jax.nn.relu(_dotT(ci, w_a) + b_a)
        return jnp.sum(o1 * w_b_row, axis=1, keepdims=True) + b_b_scalar

    bo2_s = bo2_ref[0]
    bg2_s = bg2_ref[0]
    fact_out[...] = out_head(treat_ref[...], wo1_ref[...], bo1_ref[...],
                             wo2_ref[...], bo2_s)
    targ_out[...] = out_head(treat_ref[...], wg1_ref[...], bg1_ref[...],
                             wg2_ref[...], bg2_s)
    ones = jnp.ones((_B, 1), jnp.float32)
    zeros = jnp.zeros((_B, 1), jnp.float32)
    cf0 = out_head(jnp.concatenate([ones, zeros], axis=1),
                   wo1_ref[...], bo1_ref[...], wo2_ref[...], bo2_s)
    cf1 = out_head(jnp.concatenate([zeros, ones], axis=1),
                   wo1_ref[...], bo1_ref[...], wo2_ref[...], bo2_s)
    cf_out[...] = jnp.concatenate([cf0, cf1], axis=1)


def _mlp_call(conf, pairs, par, treat, weights):
    args = (conf, pairs, par, treat) + tuple(weights)
    specs = []
    for a in args:
        if a.ndim == 1:
            specs.append(pl.BlockSpec(memory_space=pltpu.SMEM))
        else:
            specs.append(pl.BlockSpec(memory_space=pltpu.VMEM))
    return pl.pallas_call(
        _mlp_body,
        in_specs=specs,
        out_shape=[
            jax.ShapeDtypeStruct((_B, 1), jnp.float32),
            jax.ShapeDtypeStruct((_B, 2), jnp.float32),
            jax.ShapeDtypeStruct((_B, 1), jnp.float32),
            jax.ShapeDtypeStruct((_B, 2), jnp.float32),
        ],
    )(conf, pairs, par, treat, *weights)


def kernel(patient, treatment, confounders, corpus_embeddings, Wpe, bpe,
           W1, b1, g1, be1, W2, b2, g2, be2, W3, b3,
           Wo1, bo1, Wo2, bo2, Wt1, bt1, Wt2, bt2, Wg1, bg1, Wg2, bg2):
    row = lambda v: v.reshape(1, -1)
    scores_t, idx_t = _topk_call(patient, Wpe, row(bpe), corpus_embeddings)
    scores, idx = scores_t.T, idx_t.T
    idx_flat = idx.reshape(_B * _K)
    corpus_pairs = corpus_embeddings.reshape(_NDOCS // 2, 2 * _EMB)
    gathered = _gather_call(corpus_pairs, idx_flat // 2)
    pairs = gathered.reshape(_B, _K * 2 * _EMB)
    par = idx % 2
    weights = (W1, row(b1), row(g1), row(be1),
               W2, row(b2), row(g2), row(be2),
               W3, row(b3),
               Wo1, row(bo1), Wo2, bo2,
               Wt1, row(bt1), Wt2, row(bt2),
               Wg1, row(bg1), Wg2, bg2)
    factual, propensity, targeted, cf = _mlp_call(
        confounders, pairs, par, treatment, weights)
    counterfactuals = cf.reshape(_B, 2, 1)
    return (factual, propensity, targeted, counterfactuals, scores, idx)
